# baseline (speedup 1.0000x reference)
import functools

import jax
import jax.numpy as jnp
from jax import lax
from jax.experimental import pallas as pl
from jax.experimental.pallas import tpu as pltpu
from jax.experimental.pallas import tpu_sc as plsc

N = 10000
E = 320000
D = 128

NC = 2
NS = 16
NW = NC * NS
E_PER_W = E // NW
K = 128
CHUNKS = 80
EPW_PAD = K * CHUNKS
E_PAD = NW * EPW_PAD
HB_ROWS = 640
ROWS_PER_SUB = HB_ROWS // NS
ACC_ROWS = 10240
ACC_PER_SUB = ACC_ROWS // NS

_MESH = plsc.VectorSubcoreMesh(core_axis_name="c", subcore_axis_name="s")


HBINS = HB_ROWS * 16
BINS_PER_SUB = HBINS // NS


def _hist_body(dst_hbm, out_hbm, dstbuf, hist, tmp, accbuf, hist_all):
    c = lax.axis_index("c")
    s = lax.axis_index("s")
    w = c * NS + s

    zeros16 = jnp.zeros((16,), jnp.float32)

    def _zero(i, _):
        hist[pl.ds(i * 16, 16)] = zeros16
        return 0

    lax.fori_loop(0, HBINS // 16, _zero, 0)

    pltpu.sync_copy(dst_hbm.at[pl.ds(w * E_PER_W, E_PER_W)], dstbuf)

    ones16 = jnp.ones((16,), jnp.float32)

    def _acc(j, _):
        idx = dstbuf[pl.ds(j * 16, 16)]
        plsc.addupdate_scatter(hist, [idx], ones16)
        return 0

    lax.fori_loop(0, E_PER_W // 16, _acc, 0)

    pltpu.sync_copy(hist, hist_all.at[s])
    plsc.subcore_barrier()

    def _zeroacc(i, _):
        accbuf[pl.ds(i * 16, 16)] = zeros16
        return 0

    lax.fori_loop(0, BINS_PER_SUB // 16, _zeroacc, 0)

    def _combine(t, _):
        pltpu.sync_copy(hist_all.at[t, pl.ds(s * BINS_PER_SUB, BINS_PER_SUB)],
                        tmp)

        def _add(j, _):
            sl = pl.ds(j * 16, 16)
            accbuf[sl] = accbuf[sl] + tmp[sl]
            return 0

        lax.fori_loop(0, BINS_PER_SUB // 16, _add, 0)
        return 0

    lax.fori_loop(0, NS, _combine, 0)

    pltpu.sync_copy(accbuf,
                    out_hbm.at[pl.ds(c * HBINS + s * BINS_PER_SUB, BINS_PER_SUB)])


_hist_kernel = pl.kernel(
    _hist_body,
    out_type=jax.ShapeDtypeStruct((NC * HBINS,), jnp.float32),
    mesh=_MESH,
    scratch_types=[
        pltpu.VMEM((E_PER_W,), jnp.int32),
        pltpu.VMEM((HBINS,), jnp.float32),
        pltpu.VMEM((BINS_PER_SUB,), jnp.float32),
        pltpu.VMEM((BINS_PER_SUB,), jnp.float32),
        pltpu.VMEM_SHARED((NS, HBINS), jnp.float32),
    ],
    compiler_params=pltpu.CompilerParams(needs_layout_passes=False),
)


def _agg_body(y_hbm, src_hbm, dst_hbm, out_hbm, srcb, dstb0, dstb1,
              rows0, rows1, acc, sem0, sem1):
    c = lax.axis_index("c")
    s = lax.axis_index("s")
    w = c * NS + s
    base = w * EPW_PAD

    zeros16 = jnp.zeros((16,), jnp.float32)

    def _zrows(t, _):
        rows0[t // 8, pl.ds((t % 8) * 16, 16)] = zeros16
        return 0

    lax.fori_loop(0, K * 8, _zrows, 0)

    def _zacc(j, _):
        pltpu.sync_copy(rows0, acc.at[pl.ds(s * ACC_PER_SUB + j * K, K)])
        return 0

    lax.fori_loop(0, ACC_PER_SUB // K, _zacc, 0)

    pltpu.sync_copy(src_hbm.at[pl.ds(base, EPW_PAD)], srcb)
    plsc.subcore_barrier()

    pltpu.sync_copy(dst_hbm.at[pl.ds(base, K)], dstb0)
    pltpu.async_copy(y_hbm.at[srcb.at[pl.ds(0, K)]], rows0, sem0)

    def _pair(i, _):
        g = i * 2
        pltpu.make_async_copy(y_hbm.at[srcb.at[pl.ds(0, K)]], rows0,
                              sem0).wait()
        pltpu.async_copy(y_hbm.at[srcb.at[pl.ds((g + 1) * K, K)]], rows1, sem1)
        pltpu.sync_copy(dst_hbm.at[pl.ds(base + (g + 1) * K, K)], dstb1)
        pltpu.sync_copy(rows0, acc.at[dstb0], add=True)
        pltpu.make_async_copy(y_hbm.at[srcb.at[pl.ds(0, K)]], rows1,
                              sem1).wait()

        @pl.when(i < CHUNKS // 2 - 1)
        def _():
            pltpu.async_copy(y_hbm.at[srcb.at[pl.ds((g + 2) * K, K)]], rows0,
                             sem0)
            pltpu.sync_copy(dst_hbm.at[pl.ds(base + (g + 2) * K, K)], dstb0)

        pltpu.sync_copy(rows1, acc.at[dstb1], add=True)
        return 0

    lax.fori_loop(0, CHUNKS // 2, _pair, 0)
    plsc.subcore_barrier()

    pltpu.sync_copy(acc.at[pl.ds(s * ACC_PER_SUB, ACC_PER_SUB)],
                    out_hbm.at[c, pl.ds(s * ACC_PER_SUB, ACC_PER_SUB)])


_agg_kernel = pl.kernel(
    _agg_body,
    out_type=jax.ShapeDtypeStruct((NC, ACC_ROWS, D), jnp.float32),
    mesh=_MESH,
    scratch_types=[
        pltpu.VMEM((EPW_PAD,), jnp.int32),
        pltpu.VMEM((K,), jnp.int32),
        pltpu.VMEM((K,), jnp.int32),
        pltpu.VMEM((K, D), jnp.float32),
        pltpu.VMEM((K, D), jnp.float32),
        pltpu.VMEM_SHARED((ACC_ROWS, D), jnp.float32),
        pltpu.SemaphoreType.DMA,
        pltpu.SemaphoreType.DMA,
    ],
    compiler_params=pltpu.CompilerParams(needs_layout_passes=False),
)


_BLK = 1000
_GRID = N // _BLK


def _mm1_body(cnt_ref, x_ref, w_ref, y_ref, s_ref):
    deg = cnt_ref[0] + cnt_ref[1] + 1.0
    dinv = lax.rsqrt(deg)
    xw = jnp.dot(x_ref[...], w_ref[...], preferred_element_type=jnp.float32)
    y_ref[...] = dinv * xw
    s_ref[...] = (dinv * dinv) * xw


def _mm1(cnt, x, w):
    return pl.pallas_call(
        _mm1_body,
        grid=(_GRID,),
        in_specs=[
            pl.BlockSpec((2, _BLK, 1), lambda i: (0, i, 0)),
            pl.BlockSpec((_BLK, D), lambda i: (i, 0)),
            pl.BlockSpec((D, D), lambda i: (0, 0)),
        ],
        out_specs=[
            pl.BlockSpec((_BLK, D), lambda i: (i, 0)),
            pl.BlockSpec((_BLK, D), lambda i: (i, 0)),
        ],
        out_shape=[
            jax.ShapeDtypeStruct((N, D), jnp.float32),
            jax.ShapeDtypeStruct((N, D), jnp.float32),
        ],
    )(cnt, x, w)


def _mm2_body(cnt_ref, parts_ref, s1_ref, b1_ref, w_ref, y_ref, s_ref):
    deg = cnt_ref[0] + cnt_ref[1] + 1.0
    dinv = lax.rsqrt(deg)
    agg = parts_ref[0] + parts_ref[1]
    h = jnp.maximum(dinv * agg + s1_ref[...] + b1_ref[...], 0.0)
    hw = jnp.dot(h, w_ref[...], preferred_element_type=jnp.float32)
    y_ref[...] = dinv * hw
    s_ref[...] = (dinv * dinv) * hw


def _mm2(cnt, parts, s1, b1, w):
    return pl.pallas_call(
        _mm2_body,
        grid=(_GRID,),
        in_specs=[
            pl.BlockSpec((2, _BLK, 1), lambda i: (0, i, 0)),
            pl.BlockSpec((2, _BLK, D), lambda i: (0, i, 0)),
            pl.BlockSpec((_BLK, D), lambda i: (i, 0)),
            pl.BlockSpec((1, D), lambda i: (0, 0)),
            pl.BlockSpec((D, D), lambda i: (0, 0)),
        ],
        out_specs=[
            pl.BlockSpec((_BLK, D), lambda i: (i, 0)),
            pl.BlockSpec((_BLK, D), lambda i: (i, 0)),
        ],
        out_shape=[
            jax.ShapeDtypeStruct((N, D), jnp.float32),
            jax.ShapeDtypeStruct((N, D), jnp.float32),
        ],
    )(cnt, parts, s1, b1, w)


def _fin_body(cnt_ref, parts_ref, s2_ref, b2_ref, out_ref):
    deg = cnt_ref[0] + cnt_ref[1] + 1.0
    dinv = lax.rsqrt(deg)
    agg = parts_ref[0] + parts_ref[1]
    out_ref[...] = dinv * agg + s2_ref[...] + b2_ref[...]


def _fin(cnt, parts, s2, b2):
    return pl.pallas_call(
        _fin_body,
        grid=(_GRID,),
        in_specs=[
            pl.BlockSpec((2, _BLK, 1), lambda i: (0, i, 0)),
            pl.BlockSpec((2, _BLK, D), lambda i: (0, i, 0)),
            pl.BlockSpec((_BLK, D), lambda i: (i, 0)),
            pl.BlockSpec((1, D), lambda i: (0, 0)),
        ],
        out_specs=pl.BlockSpec((_BLK, D), lambda i: (i, 0)),
        out_shape=jax.ShapeDtypeStruct((N, D), jnp.float32),
    )(cnt, parts, s2, b2)


@jax.jit
def kernel(t, x, edge_index, W1, b1, W2, b2):
    src = edge_index[0]
    dst = edge_index[1]

    pad = E_PAD - E
    src_p = jnp.concatenate([src, jnp.zeros((pad,), src.dtype)])
    dead = N + jnp.arange(pad, dtype=dst.dtype) % (ACC_ROWS - N)
    dst_p = jnp.concatenate([dst, dead])

    counts = _hist_kernel(dst)
    cnt = counts.reshape(NC, HBINS, 1)[:, :N]

    y1, s1 = _mm1(cnt, x, W1)
    parts1 = _agg_kernel(y1, src_p, dst_p)
    y2, s2 = _mm2(cnt, parts1, s1, b1.reshape(1, D), W2)
    parts2 = _agg_kernel(y2, src_p, dst_p)
    return _fin(cnt, parts2, s2, b2.reshape(1, D))


# speedup vs baseline: 1.0699x; 1.0699x over previous
import functools

import jax
import jax.numpy as jnp
from jax import lax
from jax.experimental import pallas as pl
from jax.experimental.pallas import tpu as pltpu
from jax.experimental.pallas import tpu_sc as plsc

N = 10000
E = 320000
D = 128

NC = 2
NS = 16
NW = NC * NS
E_PER_W = E // NW
K = 128
CHUNKS = 80
EPW_PAD = K * CHUNKS
E_PAD = NW * EPW_PAD
HB_ROWS = 640
ROWS_PER_SUB = HB_ROWS // NS
ACC_ROWS = 10240
ACC_PER_SUB = ACC_ROWS // NS

_MESH = plsc.VectorSubcoreMesh(core_axis_name="c", subcore_axis_name="s")


HBINS = HB_ROWS * 16
BINS_PER_SUB = HBINS // NS


def _hist_body(dst_hbm, out_hbm, dstbuf, hist, tmp, accbuf, hist_all):
    c = lax.axis_index("c")
    s = lax.axis_index("s")
    w = c * NS + s

    zeros16 = jnp.zeros((16,), jnp.float32)

    def _zero(i, _):
        hist[pl.ds(i * 16, 16)] = zeros16
        return 0

    lax.fori_loop(0, HBINS // 16, _zero, 0)

    pltpu.sync_copy(dst_hbm.at[pl.ds(w * E_PER_W, E_PER_W)], dstbuf)

    ones16 = jnp.ones((16,), jnp.float32)

    def _acc(j, _):
        idx = dstbuf[pl.ds(j * 16, 16)]
        plsc.addupdate_scatter(hist, [idx], ones16)
        return 0

    lax.fori_loop(0, E_PER_W // 16, _acc, 0)

    pltpu.sync_copy(hist, hist_all.at[s])
    plsc.subcore_barrier()

    def _zeroacc(i, _):
        accbuf[pl.ds(i * 16, 16)] = zeros16
        return 0

    lax.fori_loop(0, BINS_PER_SUB // 16, _zeroacc, 0)

    def _combine(t, _):
        pltpu.sync_copy(hist_all.at[t, pl.ds(s * BINS_PER_SUB, BINS_PER_SUB)],
                        tmp)

        def _add(j, _):
            sl = pl.ds(j * 16, 16)
            accbuf[sl] = accbuf[sl] + tmp[sl]
            return 0

        lax.fori_loop(0, BINS_PER_SUB // 16, _add, 0)
        return 0

    lax.fori_loop(0, NS, _combine, 0)

    pltpu.sync_copy(accbuf,
                    out_hbm.at[pl.ds(c * HBINS + s * BINS_PER_SUB, BINS_PER_SUB)])


_hist_kernel = pl.kernel(
    _hist_body,
    out_type=jax.ShapeDtypeStruct((NC * HBINS,), jnp.float32),
    mesh=_MESH,
    scratch_types=[
        pltpu.VMEM((E_PER_W,), jnp.int32),
        pltpu.VMEM((HBINS,), jnp.float32),
        pltpu.VMEM((BINS_PER_SUB,), jnp.float32),
        pltpu.VMEM((BINS_PER_SUB,), jnp.float32),
        pltpu.VMEM_SHARED((NS, HBINS), jnp.float32),
    ],
    compiler_params=pltpu.CompilerParams(needs_layout_passes=False),
)


def _agg_body(y_hbm, src_hbm, dst_hbm, out_hbm, srcb, dstb0, dstb1,
              rows0, rows1, acc, sem0, sem1, semi0, semi1):
    c = lax.axis_index("c")
    s = lax.axis_index("s")
    w = c * NS + s
    base = w * EPW_PAD

    zeros16 = jnp.zeros((16,), jnp.float32)

    def _zrows(t, _):
        rows0[t // 8, pl.ds((t % 8) * 16, 16)] = zeros16
        return 0

    lax.fori_loop(0, K * 8, _zrows, 0)

    def _zacc(j, _):
        pltpu.sync_copy(rows0, acc.at[pl.ds(s * ACC_PER_SUB + j * K, K)])
        return 0

    lax.fori_loop(0, ACC_PER_SUB // K, _zacc, 0)

    pltpu.sync_copy(src_hbm.at[pl.ds(base, EPW_PAD)], srcb)
    plsc.subcore_barrier()

    pltpu.async_copy(dst_hbm.at[pl.ds(base, K)], dstb0, semi0)
    pltpu.async_copy(y_hbm.at[srcb.at[pl.ds(0, K)]], rows0, sem0)

    def _pair(i, _):
        g = i * 2
        pltpu.async_copy(dst_hbm.at[pl.ds(base + (g + 1) * K, K)], dstb1,
                         semi1)
        pltpu.make_async_copy(y_hbm.at[srcb.at[pl.ds(0, K)]], rows0,
                              sem0).wait()
        pltpu.async_copy(y_hbm.at[srcb.at[pl.ds((g + 1) * K, K)]], rows1, sem1)
        pltpu.make_async_copy(dst_hbm.at[pl.ds(base, K)], dstb0, semi0).wait()
        pltpu.sync_copy(rows0, acc.at[dstb0], add=True)

        @pl.when(i < CHUNKS // 2 - 1)
        def _():
            pltpu.async_copy(dst_hbm.at[pl.ds(base + (g + 2) * K, K)], dstb0,
                             semi0)

        pltpu.make_async_copy(y_hbm.at[srcb.at[pl.ds(0, K)]], rows1,
                              sem1).wait()

        @pl.when(i < CHUNKS // 2 - 1)
        def _():
            pltpu.async_copy(y_hbm.at[srcb.at[pl.ds((g + 2) * K, K)]], rows0,
                             sem0)

        pltpu.make_async_copy(dst_hbm.at[pl.ds(base, K)], dstb1, semi1).wait()
        pltpu.sync_copy(rows1, acc.at[dstb1], add=True)
        return 0

    lax.fori_loop(0, CHUNKS // 2, _pair, 0)
    plsc.subcore_barrier()

    pltpu.sync_copy(acc.at[pl.ds(s * ACC_PER_SUB, ACC_PER_SUB)],
                    out_hbm.at[c, pl.ds(s * ACC_PER_SUB, ACC_PER_SUB)])


_agg_kernel = pl.kernel(
    _agg_body,
    out_type=jax.ShapeDtypeStruct((NC, ACC_ROWS, D), jnp.float32),
    mesh=_MESH,
    scratch_types=[
        pltpu.VMEM((EPW_PAD,), jnp.int32),
        pltpu.VMEM((K,), jnp.int32),
        pltpu.VMEM((K,), jnp.int32),
        pltpu.VMEM((K, D), jnp.float32),
        pltpu.VMEM((K, D), jnp.float32),
        pltpu.VMEM_SHARED((ACC_ROWS, D), jnp.float32),
        pltpu.SemaphoreType.DMA,
        pltpu.SemaphoreType.DMA,
        pltpu.SemaphoreType.DMA,
        pltpu.SemaphoreType.DMA,
    ],
    compiler_params=pltpu.CompilerParams(needs_layout_passes=False),
)


_BLK = 1000
_GRID = N // _BLK


def _mm1_body(cnt_ref, x_ref, w_ref, y_ref, s_ref):
    deg = cnt_ref[0] + cnt_ref[1] + 1.0
    dinv = lax.rsqrt(deg)
    xw = jnp.dot(x_ref[...], w_ref[...], preferred_element_type=jnp.float32)
    y_ref[...] = dinv * xw
    s_ref[...] = (dinv * dinv) * xw


def _mm1(cnt, x, w):
    return pl.pallas_call(
        _mm1_body,
        grid=(_GRID,),
        in_specs=[
            pl.BlockSpec((2, _BLK, 1), lambda i: (0, i, 0)),
            pl.BlockSpec((_BLK, D), lambda i: (i, 0)),
            pl.BlockSpec((D, D), lambda i: (0, 0)),
        ],
        out_specs=[
            pl.BlockSpec((_BLK, D), lambda i: (i, 0)),
            pl.BlockSpec((_BLK, D), lambda i: (i, 0)),
        ],
        out_shape=[
            jax.ShapeDtypeStruct((N, D), jnp.float32),
            jax.ShapeDtypeStruct((N, D), jnp.float32),
        ],
    )(cnt, x, w)


def _mm2_body(cnt_ref, parts_ref, s1_ref, b1_ref, w_ref, y_ref, s_ref):
    deg = cnt_ref[0] + cnt_ref[1] + 1.0
    dinv = lax.rsqrt(deg)
    agg = parts_ref[0] + parts_ref[1]
    h = jnp.maximum(dinv * agg + s1_ref[...] + b1_ref[...], 0.0)
    hw = jnp.dot(h, w_ref[...], preferred_element_type=jnp.float32)
    y_ref[...] = dinv * hw
    s_ref[...] = (dinv * dinv) * hw


def _mm2(cnt, parts, s1, b1, w):
    return pl.pallas_call(
        _mm2_body,
        grid=(_GRID,),
        in_specs=[
            pl.BlockSpec((2, _BLK, 1), lambda i: (0, i, 0)),
            pl.BlockSpec((2, _BLK, D), lambda i: (0, i, 0)),
            pl.BlockSpec((_BLK, D), lambda i: (i, 0)),
            pl.BlockSpec((1, D), lambda i: (0, 0)),
            pl.BlockSpec((D, D), lambda i: (0, 0)),
        ],
        out_specs=[
            pl.BlockSpec((_BLK, D), lambda i: (i, 0)),
            pl.BlockSpec((_BLK, D), lambda i: (i, 0)),
        ],
        out_shape=[
            jax.ShapeDtypeStruct((N, D), jnp.float32),
            jax.ShapeDtypeStruct((N, D), jnp.float32),
        ],
    )(cnt, parts, s1, b1, w)


def _fin_body(cnt_ref, parts_ref, s2_ref, b2_ref, out_ref):
    deg = cnt_ref[0] + cnt_ref[1] + 1.0
    dinv = lax.rsqrt(deg)
    agg = parts_ref[0] + parts_ref[1]
    out_ref[...] = dinv * agg + s2_ref[...] + b2_ref[...]


def _fin(cnt, parts, s2, b2):
    return pl.pallas_call(
        _fin_body,
        grid=(_GRID,),
        in_specs=[
            pl.BlockSpec((2, _BLK, 1), lambda i: (0, i, 0)),
            pl.BlockSpec((2, _BLK, D), lambda i: (0, i, 0)),
            pl.BlockSpec((_BLK, D), lambda i: (i, 0)),
            pl.BlockSpec((1, D), lambda i: (0, 0)),
        ],
        out_specs=pl.BlockSpec((_BLK, D), lambda i: (i, 0)),
        out_shape=jax.ShapeDtypeStruct((N, D), jnp.float32),
    )(cnt, parts, s2, b2)


@jax.jit
def kernel(t, x, edge_index, W1, b1, W2, b2):
    src = edge_index[0]
    dst = edge_index[1]

    pad = E_PAD - E
    src_p = jnp.concatenate([src, jnp.zeros((pad,), src.dtype)])
    dead = N + jnp.arange(pad, dtype=dst.dtype) % (ACC_ROWS - N)
    dst_p = jnp.concatenate([dst, dead])

    counts = _hist_kernel(dst)
    cnt = counts.reshape(NC, HBINS, 1)[:, :N]

    y1, s1 = _mm1(cnt, x, W1)
    parts1 = _agg_kernel(y1, src_p, dst_p)
    y2, s2 = _mm2(cnt, parts1, s1, b1.reshape(1, D), W2)
    parts2 = _agg_kernel(y2, src_p, dst_p)
    return _fin(cnt, parts2, s2, b2.reshape(1, D))


# speedup vs baseline: 3.0023x; 2.8060x over previous
import functools

import jax
import jax.numpy as jnp
from jax import lax
from jax.experimental import pallas as pl
from jax.experimental.pallas import tpu as pltpu
from jax.experimental.pallas import tpu_sc as plsc

N = 10000
E = 320000
D = 128

NC = 2
NS = 16
NW = NC * NS
E_PER_W = E // NW
K = 128
CHUNKS = 80
EPW_PAD = K * CHUNKS
E_PAD = NW * EPW_PAD
HB_ROWS = 640
ROWS_PER_SUB = HB_ROWS // NS
ACC_ROWS = 10240
ACC_PER_SUB = ACC_ROWS // NS

_MESH = plsc.VectorSubcoreMesh(core_axis_name="c", subcore_axis_name="s")


HBINS = HB_ROWS * 16
BINS_PER_SUB = HBINS // NS


def _hist_body(dst_hbm, out_hbm, dstbuf, hist, tmp, accbuf, hist_all):
    c = lax.axis_index("c")
    s = lax.axis_index("s")
    w = c * NS + s

    zeros16 = jnp.zeros((16,), jnp.float32)

    def _zero(i, _):
        hist[pl.ds(i * 16, 16)] = zeros16
        return 0

    lax.fori_loop(0, HBINS // 16, _zero, 0)

    pltpu.sync_copy(dst_hbm.at[pl.ds(w * E_PER_W, E_PER_W)], dstbuf)

    ones16 = jnp.ones((16,), jnp.float32)

    def _acc(j, _):
        idx = dstbuf[pl.ds(j * 16, 16)]
        plsc.addupdate_scatter(hist, [idx], ones16)
        return 0

    lax.fori_loop(0, E_PER_W // 16, _acc, 0)

    pltpu.sync_copy(hist, hist_all.at[s])
    plsc.subcore_barrier()

    def _zeroacc(i, _):
        accbuf[pl.ds(i * 16, 16)] = zeros16
        return 0

    lax.fori_loop(0, BINS_PER_SUB // 16, _zeroacc, 0)

    def _combine(t, _):
        pltpu.sync_copy(hist_all.at[t, pl.ds(s * BINS_PER_SUB, BINS_PER_SUB)],
                        tmp)

        def _add(j, _):
            sl = pl.ds(j * 16, 16)
            accbuf[sl] = accbuf[sl] + tmp[sl]
            return 0

        lax.fori_loop(0, BINS_PER_SUB // 16, _add, 0)
        return 0

    lax.fori_loop(0, NS, _combine, 0)

    pltpu.sync_copy(accbuf,
                    out_hbm.at[pl.ds(c * HBINS + s * BINS_PER_SUB, BINS_PER_SUB)])


_hist_kernel = pl.kernel(
    _hist_body,
    out_type=jax.ShapeDtypeStruct((NC * HBINS,), jnp.float32),
    mesh=_MESH,
    scratch_types=[
        pltpu.VMEM((E_PER_W,), jnp.int32),
        pltpu.VMEM((HBINS,), jnp.float32),
        pltpu.VMEM((BINS_PER_SUB,), jnp.float32),
        pltpu.VMEM((BINS_PER_SUB,), jnp.float32),
        pltpu.VMEM_SHARED((NS, HBINS), jnp.float32),
    ],
    compiler_params=pltpu.CompilerParams(needs_layout_passes=False),
)


def _agg_body(y_hbm, src_hbm, dst_hbm, out_hbm, srcb, dstb0, dstb1,
              rows0, rows1, acc, sem0, sem1, semi0, semi1):
    c = lax.axis_index("c")
    s = lax.axis_index("s")
    w = c * NS + s
    base = w * EPW_PAD

    zeros16 = jnp.zeros((16,), jnp.float32)

    def _zrows(t, _):
        rows0[t // 8, pl.ds((t % 8) * 16, 16)] = zeros16
        return 0

    lax.fori_loop(0, K * 8, _zrows, 0)

    def _zacc(j, _):
        pltpu.sync_copy(rows0, acc.at[pl.ds(s * ACC_PER_SUB + j * K, K)])
        return 0

    lax.fori_loop(0, ACC_PER_SUB // K, _zacc, 0)

    pltpu.sync_copy(src_hbm.at[pl.ds(base, EPW_PAD)], srcb)
    plsc.subcore_barrier()

    yc = y_hbm.at[c]
    pltpu.async_copy(dst_hbm.at[pl.ds(base, K)], dstb0, semi0)
    pltpu.async_copy(yc.at[srcb.at[pl.ds(0, K)]], rows0, sem0)

    def _pair(i, _):
        g = i * 2
        pltpu.async_copy(dst_hbm.at[pl.ds(base + (g + 1) * K, K)], dstb1,
                         semi1)
        pltpu.make_async_copy(yc.at[srcb.at[pl.ds(0, K)]], rows0,
                              sem0).wait()
        pltpu.async_copy(yc.at[srcb.at[pl.ds((g + 1) * K, K)]], rows1, sem1)
        pltpu.make_async_copy(dst_hbm.at[pl.ds(base, K)], dstb0, semi0).wait()
        pltpu.sync_copy(rows0, acc.at[dstb0], add=True)

        @pl.when(i < CHUNKS // 2 - 1)
        def _():
            pltpu.async_copy(dst_hbm.at[pl.ds(base + (g + 2) * K, K)], dstb0,
                             semi0)

        pltpu.make_async_copy(yc.at[srcb.at[pl.ds(0, K)]], rows1,
                              sem1).wait()

        @pl.when(i < CHUNKS // 2 - 1)
        def _():
            pltpu.async_copy(yc.at[srcb.at[pl.ds((g + 2) * K, K)]], rows0,
                             sem0)

        pltpu.make_async_copy(dst_hbm.at[pl.ds(base, K)], dstb1, semi1).wait()
        pltpu.sync_copy(rows1, acc.at[dstb1], add=True)
        return 0

    lax.fori_loop(0, CHUNKS // 2, _pair, 0)
    plsc.subcore_barrier()

    pltpu.sync_copy(acc.at[pl.ds(s * ACC_PER_SUB, ACC_PER_SUB)],
                    out_hbm.at[c, pl.ds(s * ACC_PER_SUB, ACC_PER_SUB)])


_agg_kernel = pl.kernel(
    _agg_body,
    out_type=jax.ShapeDtypeStruct((NC, ACC_ROWS, D), jnp.float32),
    mesh=_MESH,
    scratch_types=[
        pltpu.VMEM((EPW_PAD,), jnp.int32),
        pltpu.VMEM((K,), jnp.int32),
        pltpu.VMEM((K,), jnp.int32),
        pltpu.VMEM((K, D), jnp.float32),
        pltpu.VMEM((K, D), jnp.float32),
        pltpu.VMEM_SHARED((ACC_ROWS, D), jnp.float32),
        pltpu.SemaphoreType.DMA,
        pltpu.SemaphoreType.DMA,
        pltpu.SemaphoreType.DMA,
        pltpu.SemaphoreType.DMA,
    ],
    compiler_params=pltpu.CompilerParams(needs_layout_passes=False),
)


_BLK = 1000
_GRID = N // _BLK


def _mm1_body(cnt_ref, x_ref, w_ref, y_ref, s_ref):
    deg = cnt_ref[0] + cnt_ref[1] + 1.0
    dinv = lax.rsqrt(deg)
    xw = jnp.dot(x_ref[...], w_ref[...], preferred_element_type=jnp.float32)
    y = dinv * xw
    y_ref[0] = y
    y_ref[1] = y
    s_ref[...] = (dinv * dinv) * xw


def _mm1(cnt, x, w):
    return pl.pallas_call(
        _mm1_body,
        grid=(_GRID,),
        in_specs=[
            pl.BlockSpec((2, _BLK, 1), lambda i: (0, i, 0)),
            pl.BlockSpec((_BLK, D), lambda i: (i, 0)),
            pl.BlockSpec((D, D), lambda i: (0, 0)),
        ],
        out_specs=[
            pl.BlockSpec((2, _BLK, D), lambda i: (0, i, 0)),
            pl.BlockSpec((_BLK, D), lambda i: (i, 0)),
        ],
        out_shape=[
            jax.ShapeDtypeStruct((2, N, D), jnp.float32),
            jax.ShapeDtypeStruct((N, D), jnp.float32),
        ],
    )(cnt, x, w)


def _mm2_body(cnt_ref, parts_ref, s1_ref, b1_ref, w_ref, y_ref, s_ref):
    deg = cnt_ref[0] + cnt_ref[1] + 1.0
    dinv = lax.rsqrt(deg)
    agg = parts_ref[0] + parts_ref[1]
    h = jnp.maximum(dinv * agg + s1_ref[...] + b1_ref[...], 0.0)
    hw = jnp.dot(h, w_ref[...], preferred_element_type=jnp.float32)
    y = dinv * hw
    y_ref[0] = y
    y_ref[1] = y
    s_ref[...] = (dinv * dinv) * hw


def _mm2(cnt, parts, s1, b1, w):
    return pl.pallas_call(
        _mm2_body,
        grid=(_GRID,),
        in_specs=[
            pl.BlockSpec((2, _BLK, 1), lambda i: (0, i, 0)),
            pl.BlockSpec((2, _BLK, D), lambda i: (0, i, 0)),
            pl.BlockSpec((_BLK, D), lambda i: (i, 0)),
            pl.BlockSpec((1, D), lambda i: (0, 0)),
            pl.BlockSpec((D, D), lambda i: (0, 0)),
        ],
        out_specs=[
            pl.BlockSpec((2, _BLK, D), lambda i: (0, i, 0)),
            pl.BlockSpec((_BLK, D), lambda i: (i, 0)),
        ],
        out_shape=[
            jax.ShapeDtypeStruct((2, N, D), jnp.float32),
            jax.ShapeDtypeStruct((N, D), jnp.float32),
        ],
    )(cnt, parts, s1, b1, w)


def _fin_body(cnt_ref, parts_ref, s2_ref, b2_ref, out_ref):
    deg = cnt_ref[0] + cnt_ref[1] + 1.0
    dinv = lax.rsqrt(deg)
    agg = parts_ref[0] + parts_ref[1]
    out_ref[...] = dinv * agg + s2_ref[...] + b2_ref[...]


def _fin(cnt, parts, s2, b2):
    return pl.pallas_call(
        _fin_body,
        grid=(_GRID,),
        in_specs=[
            pl.BlockSpec((2, _BLK, 1), lambda i: (0, i, 0)),
            pl.BlockSpec((2, _BLK, D), lambda i: (0, i, 0)),
            pl.BlockSpec((_BLK, D), lambda i: (i, 0)),
            pl.BlockSpec((1, D), lambda i: (0, 0)),
        ],
        out_specs=pl.BlockSpec((_BLK, D), lambda i: (i, 0)),
        out_shape=jax.ShapeDtypeStruct((N, D), jnp.float32),
    )(cnt, parts, s2, b2)


@jax.jit
def kernel(t, x, edge_index, W1, b1, W2, b2):
    src = edge_index[0]
    dst = edge_index[1]

    pad = E_PAD - E
    src_p = jnp.concatenate([src, jnp.zeros((pad,), src.dtype)])
    dead = N + jnp.arange(pad, dtype=dst.dtype) % (ACC_ROWS - N)
    dst_p = jnp.concatenate([dst, dead])

    counts = _hist_kernel(dst)
    cnt = counts.reshape(NC, HBINS, 1)[:, :N]

    y1, s1 = _mm1(cnt, x, W1)
    parts1 = _agg_kernel(y1, src_p, dst_p)
    y2, s2 = _mm2(cnt, parts1, s1, b1.reshape(1, D), W2)
    parts2 = _agg_kernel(y2, src_p, dst_p)
    return _fin(cnt, parts2, s2, b2.reshape(1, D))
